# baseline (speedup 1.0000x reference)
import jax
import jax.numpy as jnp
from jax import lax
from jax.experimental import pallas as pl
from jax.experimental.pallas import tpu as pltpu
from jax.experimental.pallas import tpu_sc as plsc

B = 16384
T = 50
E = 32
V = 1000000
NC, NS = 2, 16
NW = NC * NS
UNIT = 1024
NUNit = B * T // UNIT
PWU = NUNit // NW
JPT = B // UNIT


def _body(tbl, dt3, out, idx0, idx1, rows0, rows1, tb, sg0, sg1, ss):
    wid = lax.axis_index("s") * NC + lax.axis_index("c")
    idxb = (idx0, idx1)
    rows = (rows0, rows1)
    sg = (sg0, sg1)
    tblr = tbl
    iota = jax.lax.iota(jnp.int32, 16)

    def load_idx(u, b):
        t = u // JPT
        j = u % JPT
        pltpu.sync_copy(dt3.at[t, pl.ds(j * 8, 8)], idxb[b])

    def fire(b):
        for r in range(8):
            pltpu.async_copy(
                tblr.at[idxb[b].at[r]],
                rows[b].at[pl.ds(r * 128, 128)],
                sg[b],
            )

    def drain_gather(b):
        pltpu.make_async_copy(tblr.at[pl.ds(0, UNIT)], rows[b], sg[b]).wait()

    def drain_store():
        pltpu.make_async_copy(out.at[pl.ds(0, UNIT * E)], tb, ss).wait()

    def transpose(b):
        @plsc.parallel_loop(0, 64, unroll=8)
        def step(lv):
            row_idx = lv * 16 + iota
            for trs in range(E):
                col = jnp.full((16,), trs, jnp.int32)
                v = plsc.load_gather(rows[b], [row_idx, col])
                tr_, s_ = divmod(trs, 8)
                off = tr_ * 8192 + s_ * 128 + (lv >> 3) * 1024 + (lv & 7) * 16
                tb[pl.ds(off, 16)] = v

    def store(u):
        t = u // JPT
        j = u % JPT
        for tr in range(4):
            pltpu.async_copy(
                tb.at[pl.ds(tr * 8192, 8192)],
                out.at[pl.ds(((t * 4 + tr) * 128 + j * 8) * 1024, 8192)],
                ss,
            )

    u0 = wid * PWU
    load_idx(u0, 0)
    fire(0)
    store(u0)

    def pair(g, carry):
        a = u0 + 2 * g
        load_idx(a + 1, 1)
        fire(1)
        drain_gather(0)
        drain_store()
        transpose(0)
        store(a)
        load_idx(a + 2, 0)
        fire(0)
        drain_gather(1)
        drain_store()
        transpose(1)
        store(a + 1)
        return carry

    lax.fori_loop(0, PWU // 2, pair, 0)
    drain_gather(0)
    drain_store()
    transpose(0)
    store(u0 + PWU - 1)
    drain_store()


@jax.jit
def _emb_lookup(tbl, dt3):
    mesh = plsc.VectorSubcoreMesh(core_axis_name="c", subcore_axis_name="s")
    f = pl.kernel(
        _body,
        out_type=jax.ShapeDtypeStruct((B * T * E,), jnp.float32),
        mesh=mesh,
        scratch_types=[
            pltpu.VMEM((8, 128), jnp.int32),
            pltpu.VMEM((8, 128), jnp.int32),
            pltpu.VMEM((UNIT, E), jnp.float32),
            pltpu.VMEM((UNIT, E), jnp.float32),
            pltpu.VMEM((UNIT * E,), jnp.float32),
            pltpu.SemaphoreType.DMA,
            pltpu.SemaphoreType.DMA,
            pltpu.SemaphoreType.DMA,
        ],
        compiler_params=pltpu.CompilerParams(
            use_tc_tiling_on_sc=False, needs_layout_passes=False
        ),
    )
    return f(tbl, dt3)


def kernel(data, iemb):
    tbl128 = lax.optimization_barrier(iemb.reshape(V * E // 128, 128))
    tbl = tbl128.reshape(V, E)
    dt3 = data.T.reshape(T, B // 128, 128)
    oflat = _emb_lookup(tbl, dt3)
    o5 = oflat.reshape(T, E // 8, B // 128, 8, 128)
    return o5.transpose(2, 4, 0, 1, 3).reshape(B, T, E)


# speedup vs baseline: 1.2875x; 1.2875x over previous
import jax
import jax.numpy as jnp
from jax import lax
from jax.experimental import pallas as pl
from jax.experimental.pallas import tpu as pltpu
from jax.experimental.pallas import tpu_sc as plsc

B = 16384
T = 50
E = 32
V = 1000000
NC, NS = 2, 16
NW = NC * NS
UNIT = 1024
NUNit = B * T // UNIT
PWU = NUNit // NW
JPT = B // UNIT


def _body(tbl, dt3, out, idx0, idx1, rows0, rows1, tb, sg0, sg1, ss):
    wid = lax.axis_index("s") * NC + lax.axis_index("c")
    idxb = (idx0, idx1)
    rows = (rows0, rows1)
    sg = (sg0, sg1)
    tblr = tbl
    iota = jax.lax.iota(jnp.int32, 16)
    lane_k = iota >> 2
    lane_d = iota & 3
    lane_dst = lane_d * 128 + lane_k

    def load_idx(u, b):
        t = u // JPT
        j = u % JPT
        pltpu.sync_copy(dt3.at[t, pl.ds(j * 8, 8)], idxb[b])

    def fire(b):
        for r in range(8):
            pltpu.async_copy(
                tblr.at[idxb[b].at[r]],
                rows[b].at[pl.ds(r * 128, 128)],
                sg[b],
            )

    def drain_gather(b):
        pltpu.make_async_copy(tblr.at[pl.ds(0, UNIT)], rows[b], sg[b]).wait()

    def drain_store():
        pltpu.make_async_copy(out.at[pl.ds(0, UNIT * E)], tb, ss).wait()

    def transpose(b):
        @plsc.parallel_loop(0, 64, unroll=4)
        def step(lv):
            c = lv >> 3
            lqa = lv & 7
            for inner in range(E):
                lqb = inner >> 3
                tr_ = (inner >> 1) & 3
                e0h = inner & 1
                row_base = c * 128 + lqa * 16 + lqb * 4
                e0 = tr_ * 8 + e0h * 4
                row_v = row_base + lane_k
                col_v = e0 + lane_d
                v = plsc.load_gather(rows[b], [row_v, col_v])
                dstb = (tr_ * 8192 + e0h * 512 + lqb * 4) + c * 1024 + lqa * 16
                plsc.store_scatter(tb, [dstb + lane_dst], v)

    def store(u):
        t = u // JPT
        j = u % JPT
        for tr in range(4):
            pltpu.async_copy(
                tb.at[pl.ds(tr * 8192, 8192)],
                out.at[pl.ds(((t * 4 + tr) * 128 + j * 8) * 1024, 8192)],
                ss,
            )

    u0 = wid * PWU
    load_idx(u0, 0)
    fire(0)
    store(u0)

    def pair(g, carry):
        a = u0 + 2 * g
        load_idx(a + 1, 1)
        fire(1)
        drain_gather(0)
        drain_store()
        transpose(0)
        store(a)
        load_idx(a + 2, 0)
        fire(0)
        drain_gather(1)
        drain_store()
        transpose(1)
        store(a + 1)
        return carry

    lax.fori_loop(0, PWU // 2, pair, 0)
    drain_gather(0)
    drain_store()
    transpose(0)
    store(u0 + PWU - 1)
    drain_store()


@jax.jit
def _emb_lookup(tbl, dt3):
    mesh = plsc.VectorSubcoreMesh(core_axis_name="c", subcore_axis_name="s")
    f = pl.kernel(
        _body,
        out_type=jax.ShapeDtypeStruct((B * T * E,), jnp.float32),
        mesh=mesh,
        scratch_types=[
            pltpu.VMEM((8, 128), jnp.int32),
            pltpu.VMEM((8, 128), jnp.int32),
            pltpu.VMEM((UNIT, E), jnp.float32),
            pltpu.VMEM((UNIT, E), jnp.float32),
            pltpu.VMEM((UNIT * E,), jnp.float32),
            pltpu.SemaphoreType.DMA,
            pltpu.SemaphoreType.DMA,
            pltpu.SemaphoreType.DMA,
        ],
        compiler_params=pltpu.CompilerParams(
            use_tc_tiling_on_sc=False, needs_layout_passes=False
        ),
    )
    return f(tbl, dt3)


def kernel(data, iemb):
    tbl128 = lax.optimization_barrier(iemb.reshape(V * E // 128, 128))
    tbl = tbl128.reshape(V, E)
    dt3 = data.T.reshape(T, B // 128, 128)
    oflat = _emb_lookup(tbl, dt3)
    o5 = oflat.reshape(T, E // 8, B // 128, 8, 128)
    return o5.transpose(2, 4, 0, 1, 3).reshape(B, T, E)


# speedup vs baseline: 1.5253x; 1.1847x over previous
import jax
import jax.numpy as jnp
from jax import lax
from jax.experimental import pallas as pl
from jax.experimental.pallas import tpu as pltpu
from jax.experimental.pallas import tpu_sc as plsc

B = 16384
T = 50
E = 32
V = 1000000
NC, NS = 2, 16
NW = NC * NS
UNIT = 1024
NUNit = B * T // UNIT
PWU = NUNit // NW
JPT = B // UNIT


def _body(tbl, dt3, out, idx0, idx1, rows0, rows1, tb, sg0, sg1, ss):
    wid = lax.axis_index("s") * NC + lax.axis_index("c")
    idxb = (idx0, idx1)
    rows = (rows0, rows1)
    sg = (sg0, sg1)
    tblr = tbl
    iota = jax.lax.iota(jnp.int32, 16)
    lane_k = iota >> 2
    lane_d = iota & 3
    lane_dst = lane_d * 128 + lane_k

    def load_idx(u, b):
        t = u // JPT
        j = u % JPT
        pltpu.sync_copy(dt3.at[t, pl.ds(j * 8, 8)], idxb[b])

    def fire(b):
        for r in range(8):
            pltpu.async_copy(
                tblr.at[idxb[b].at[r]],
                rows[b].at[pl.ds(r * 128, 128)],
                sg[b],
            )

    def drain_gather(b):
        pltpu.make_async_copy(tblr.at[pl.ds(0, UNIT)], rows[b], sg[b]).wait()

    def drain_store():
        pltpu.make_async_copy(out.at[pl.ds(0, UNIT * E)], tb, ss).wait()

    def transpose(b):
        @plsc.parallel_loop(0, 64, unroll=8)
        def step(lv):
            c = lv >> 3
            lqa = lv & 7
            for inner in range(E):
                lqb = inner >> 3
                tr_ = (inner >> 1) & 3
                e0h = inner & 1
                row_base = c * 128 + lqa * 16 + lqb * 4
                e0 = tr_ * 8 + e0h * 4
                row_v = row_base + lane_k
                col_v = e0 + lane_d
                v = plsc.load_gather(rows[b], [row_v, col_v])
                dstb = (tr_ * 8192 + e0h * 512 + lqb * 4) + c * 1024 + lqa * 16
                plsc.store_scatter(tb, [dstb + lane_dst], v)

    def store(u):
        t = u // JPT
        j = u % JPT
        for tr in range(4):
            pltpu.async_copy(
                tb.at[pl.ds(tr * 8192, 8192)],
                out.at[pl.ds(((t * 4 + tr) * 128 + j * 8) * 1024, 8192)],
                ss,
            )

    u0 = wid * PWU
    load_idx(u0, 0)
    fire(0)
    store(u0)

    def pair(g, carry):
        a = u0 + 2 * g
        load_idx(a + 1, 1)
        fire(1)
        drain_gather(0)
        drain_store()
        transpose(0)
        store(a)
        load_idx(a + 2, 0)
        fire(0)
        drain_gather(1)
        drain_store()
        transpose(1)
        store(a + 1)
        return carry

    lax.fori_loop(0, PWU // 2, pair, 0)
    drain_gather(0)
    drain_store()
    transpose(0)
    store(u0 + PWU - 1)
    drain_store()


@jax.jit
def _emb_lookup(tbl, dt3):
    mesh = plsc.VectorSubcoreMesh(core_axis_name="c", subcore_axis_name="s")
    f = pl.kernel(
        _body,
        out_type=jax.ShapeDtypeStruct((B * T * E,), jnp.float32),
        mesh=mesh,
        scratch_types=[
            pltpu.VMEM((8, 128), jnp.int32),
            pltpu.VMEM((8, 128), jnp.int32),
            pltpu.VMEM((UNIT, E), jnp.float32),
            pltpu.VMEM((UNIT, E), jnp.float32),
            pltpu.VMEM((UNIT * E,), jnp.float32),
            pltpu.SemaphoreType.DMA,
            pltpu.SemaphoreType.DMA,
            pltpu.SemaphoreType.DMA,
        ],
        compiler_params=pltpu.CompilerParams(
            use_tc_tiling_on_sc=False, needs_layout_passes=False
        ),
    )
    return f(tbl, dt3)


def kernel(data, iemb):
    tbl128 = lax.optimization_barrier(iemb.reshape(V * E // 128, 128))
    tbl = tbl128.reshape(V, E)
    dt3 = data.T.reshape(T, B // 128, 128)
    oflat = _emb_lookup(tbl, dt3)
    o5 = oflat.reshape(T, E // 8, B // 128, 8, 128)
    return o5.transpose(2, 4, 0, 1, 3).reshape(B, T, E)


# speedup vs baseline: 1.5296x; 1.0028x over previous
import jax
import jax.numpy as jnp
from jax import lax
from jax.experimental import pallas as pl
from jax.experimental.pallas import tpu as pltpu
from jax.experimental.pallas import tpu_sc as plsc

B = 16384
T = 50
E = 32
V = 1000000
NC, NS = 2, 16
NW = NC * NS
UNIT = 1024
NUNit = B * T // UNIT
PWU = NUNit // NW
JPT = B // UNIT


def _body(tbl, dt3, out, idx0, idx1, rows0, rows1, tb, sg0, sg1, ss):
    wid = lax.axis_index("s") * NC + lax.axis_index("c")
    idxb = (idx0, idx1)
    rows = (rows0, rows1)
    sg = (sg0, sg1)
    tblr = tbl
    iota = jax.lax.iota(jnp.int32, 16)
    lane_k = iota >> 2
    lane_d = iota & 3
    lane_dst = lane_d * 128 + lane_k

    def load_idx(u, b):
        t = u // JPT
        j = u % JPT
        pltpu.sync_copy(dt3.at[t, pl.ds(j * 8, 8)], idxb[b])

    def fire(b):
        for r in range(8):
            pltpu.async_copy(
                tblr.at[idxb[b].at[r]],
                rows[b].at[pl.ds(r * 128, 128)],
                sg[b],
            )

    def drain_gather(b):
        pltpu.make_async_copy(tblr.at[pl.ds(0, UNIT)], rows[b], sg[b]).wait()

    def drain_store():
        pltpu.make_async_copy(out.at[pl.ds(0, UNIT * E)], tb, ss).wait()

    def transpose(b):
        @plsc.parallel_loop(0, 64, unroll=16)
        def step(lv):
            c = lv >> 3
            lqa = lv & 7
            for inner in range(E):
                lqb = inner >> 3
                tr_ = (inner >> 1) & 3
                e0h = inner & 1
                row_base = c * 128 + lqa * 16 + lqb * 4
                e0 = tr_ * 8 + e0h * 4
                row_v = row_base + lane_k
                col_v = e0 + lane_d
                v = plsc.load_gather(rows[b], [row_v, col_v])
                dstb = (tr_ * 8192 + e0h * 512 + lqb * 4) + c * 1024 + lqa * 16
                plsc.store_scatter(tb, [dstb + lane_dst], v)

    def store(u):
        t = u // JPT
        j = u % JPT
        for tr in range(4):
            pltpu.async_copy(
                tb.at[pl.ds(tr * 8192, 8192)],
                out.at[pl.ds(((t * 4 + tr) * 128 + j * 8) * 1024, 8192)],
                ss,
            )

    u0 = wid * PWU
    load_idx(u0, 0)
    fire(0)
    store(u0)

    def pair(g, carry):
        a = u0 + 2 * g
        load_idx(a + 1, 1)
        fire(1)
        drain_gather(0)
        drain_store()
        transpose(0)
        store(a)
        load_idx(a + 2, 0)
        fire(0)
        drain_gather(1)
        drain_store()
        transpose(1)
        store(a + 1)
        return carry

    lax.fori_loop(0, PWU // 2, pair, 0)
    drain_gather(0)
    drain_store()
    transpose(0)
    store(u0 + PWU - 1)
    drain_store()


@jax.jit
def _emb_lookup(tbl, dt3):
    mesh = plsc.VectorSubcoreMesh(core_axis_name="c", subcore_axis_name="s")
    f = pl.kernel(
        _body,
        out_type=jax.ShapeDtypeStruct((B * T * E,), jnp.float32),
        mesh=mesh,
        scratch_types=[
            pltpu.VMEM((8, 128), jnp.int32),
            pltpu.VMEM((8, 128), jnp.int32),
            pltpu.VMEM((UNIT, E), jnp.float32),
            pltpu.VMEM((UNIT, E), jnp.float32),
            pltpu.VMEM((UNIT * E,), jnp.float32),
            pltpu.SemaphoreType.DMA,
            pltpu.SemaphoreType.DMA,
            pltpu.SemaphoreType.DMA,
        ],
        compiler_params=pltpu.CompilerParams(
            use_tc_tiling_on_sc=False, needs_layout_passes=False
        ),
    )
    return f(tbl, dt3)


def kernel(data, iemb):
    tbl128 = lax.optimization_barrier(iemb.reshape(V * E // 128, 128))
    tbl = tbl128.reshape(V, E)
    dt3 = data.T.reshape(T, B // 128, 128)
    oflat = _emb_lookup(tbl, dt3)
    o5 = oflat.reshape(T, E // 8, B // 128, 8, 128)
    return o5.transpose(2, 4, 0, 1, 3).reshape(B, T, E)
